# Initial kernel scaffold; baseline (speedup 1.0000x reference)
import jax
import jax.numpy as jnp
from jax.experimental import pallas as pl


def kernel(x, h, W_ih_0, W_hh_0, b_ih_0, b_hh_0, W_ih_1, W_hh_1, b_ih_1, b_hh_1):
    raise NotImplementedError("write your pallas kernel here")


# speedup vs baseline: 3.0936x; 3.0936x over previous
import jax
import jax.numpy as jnp
from jax.experimental import pallas as pl
from jax.experimental.pallas import tpu as pltpu

SEQ_LEN = 8192
HID = 2048
CHUNK = 128
NCHUNK = SEQ_LEN // CHUNK


def _split_hi_lo(w):
    hi = w.astype(jnp.bfloat16)
    lo = (w - hi.astype(jnp.float32)).astype(jnp.bfloat16)
    return hi, lo


def _dot3(v, w_hi_ref, w_lo_ref):
    v_hi = v.astype(jnp.bfloat16)
    v_lo = (v - v_hi.astype(jnp.float32)).astype(jnp.bfloat16)
    lhs = jnp.concatenate([v_hi, v_lo], axis=0)
    a = jnp.dot(lhs, w_hi_ref[...], preferred_element_type=jnp.float32)
    b = jnp.dot(v_hi, w_lo_ref[...], preferred_element_type=jnp.float32)
    return a[0:1, :] + a[1:2, :] + b


def _rnn_body(x_ref, h_ref, w0_ref, b0i_ref, b0h_ref,
              wt0h_ref, wt0l_ref, wt1ih_ref, wt1il_ref,
              b1i_ref, b1h_ref, wt1hh_ref, wt1hl_ref,
              out_ref, hs0_ref, xp1_ref, hc_ref):
    c = pl.program_id(0)

    @pl.when(c == 0)
    def _():
        hc_ref[...] = h_ref[...]

    base = c * CHUNK
    w0 = w0_ref[...]
    b0 = b0i_ref[...] + b0h_ref[...]

    def step0(t, h):
        xt = x_ref[0, base + t]
        pre = xt * w0 + b0 + _dot3(h, wt0h_ref, wt0l_ref)
        hn = jnp.tanh(pre)
        hs0_ref[pl.ds(t, 1), :] = hn
        return hn

    h0 = jax.lax.fori_loop(0, CHUNK, step0, hc_ref[0:1, :])
    hc_ref[0:1, :] = h0

    hs = hs0_ref[...]
    hs_hi = hs.astype(jnp.bfloat16)
    hs_lo = (hs - hs_hi.astype(jnp.float32)).astype(jnp.bfloat16)
    lhs = jnp.concatenate([hs_hi, hs_lo], axis=0)
    a = jnp.dot(lhs, wt1ih_ref[...], preferred_element_type=jnp.float32)
    b = jnp.dot(hs_hi, wt1il_ref[...], preferred_element_type=jnp.float32)
    xp1_ref[...] = (a[0:CHUNK, :] + a[CHUNK:2 * CHUNK, :] + b
                    + b1i_ref[...] + b1h_ref[...])

    def step1(t, h):
        pre = xp1_ref[pl.ds(t, 1), :] + _dot3(h, wt1hh_ref, wt1hl_ref)
        return jnp.tanh(pre)

    h1 = jax.lax.fori_loop(0, CHUNK, step1, hc_ref[1:2, :])
    hc_ref[1:2, :] = h1

    @pl.when(c == NCHUNK - 1)
    def _():
        out_ref[...] = hc_ref[...]


def kernel(x, h, W_ih_0, W_hh_0, b_ih_0, b_hh_0, W_ih_1, W_hh_1, b_ih_1, b_hh_1):
    vmem = lambda: pl.BlockSpec(memory_space=pltpu.VMEM)
    wt0_hi, wt0_lo = _split_hi_lo(W_hh_0.T)
    wt1i_hi, wt1i_lo = _split_hi_lo(W_ih_1.T)
    wt1h_hi, wt1h_lo = _split_hi_lo(W_hh_1.T)
    out = pl.pallas_call(
        _rnn_body,
        grid=(NCHUNK,),
        in_specs=[
            pl.BlockSpec(memory_space=pltpu.SMEM),
            vmem(),
            vmem(),
            vmem(),
            vmem(),
            vmem(), vmem(),
            vmem(), vmem(),
            vmem(),
            vmem(),
            vmem(), vmem(),
        ],
        out_specs=vmem(),
        out_shape=jax.ShapeDtypeStruct((2, HID), jnp.float32),
        scratch_shapes=[
            pltpu.VMEM((CHUNK, HID), jnp.float32),
            pltpu.VMEM((CHUNK, HID), jnp.float32),
            pltpu.VMEM((2, HID), jnp.float32),
        ],
        compiler_params=pltpu.CompilerParams(
            dimension_semantics=("arbitrary",),
            vmem_limit_bytes=100 * 1024 * 1024,
        ),
    )(
        x,
        h.reshape(2, HID),
        W_ih_0.T,
        b_ih_0.reshape(1, HID),
        b_hh_0.reshape(1, HID),
        wt0_hi, wt0_lo,
        wt1i_hi, wt1i_lo,
        b_ih_1.reshape(1, HID),
        b_hh_1.reshape(1, HID),
        wt1h_hi, wt1h_lo,
    )
    return out.reshape(2, 1, HID)


# speedup vs baseline: 4.2223x; 1.3649x over previous
import jax
import jax.numpy as jnp
from jax.experimental import pallas as pl
from jax.experimental.pallas import tpu as pltpu

SEQ_LEN = 8192
HID = 2048
CHUNK = 128
NCHUNK = SEQ_LEN // CHUNK


LO_SCALE = 4096.0
INV_LO_SCALE = 1.0 / LO_SCALE


def _split_hi_lo(w):
    hi = w.astype(jnp.bfloat16)
    lo = ((w - hi.astype(jnp.float32)) * LO_SCALE).astype(jnp.float8_e4m3fn)
    return hi, lo


def _dot3(v, w_hi_ref, w_lo_ref):
    v_hi = v.astype(jnp.bfloat16)
    v_lo = (v - v_hi.astype(jnp.float32)).astype(jnp.bfloat16)
    lhs = jnp.concatenate([v_hi, v_lo], axis=0)
    a = jnp.dot(lhs, w_hi_ref[...], preferred_element_type=jnp.float32)
    b = jnp.dot(v_hi.astype(jnp.float8_e4m3fn), w_lo_ref[...],
                preferred_element_type=jnp.float32)
    return a[0:1, :] + a[1:2, :] + b * INV_LO_SCALE


def _rnn_body(x_ref, h_ref, w0_ref, b0i_ref, b0h_ref,
              wt0h_ref, wt0l_ref, wt1ih_ref, wt1il_ref,
              b1i_ref, b1h_ref, wt1hh_ref, wt1hl_ref,
              out_ref, hs0_ref, xp1_ref, hc_ref):
    c = pl.program_id(0)

    @pl.when(c == 0)
    def _():
        hc_ref[...] = h_ref[...]

    base = c * CHUNK
    w0 = w0_ref[...]
    b0 = b0i_ref[...] + b0h_ref[...]

    def step0(t, h):
        xt = x_ref[0, base + t]
        pre = xt * w0 + b0 + _dot3(h, wt0h_ref, wt0l_ref)
        hn = jnp.tanh(pre)
        hs0_ref[pl.ds(t, 1), :] = hn
        return hn

    h0 = jax.lax.fori_loop(0, CHUNK, step0, hc_ref[0:1, :], unroll=2)
    hc_ref[0:1, :] = h0

    hs = hs0_ref[...]
    hs_hi = hs.astype(jnp.bfloat16)
    xp1_ref[...] = (jnp.dot(hs_hi, wt1ih_ref[...],
                            preferred_element_type=jnp.float32)
                    + b1i_ref[...] + b1h_ref[...])
    hs_lo = (hs - hs_hi.astype(jnp.float32)).astype(jnp.bfloat16)
    xp1_ref[...] += jnp.dot(hs_lo, wt1ih_ref[...],
                            preferred_element_type=jnp.float32)
    xp1_ref[...] += jnp.dot(hs_hi.astype(jnp.float8_e4m3fn), wt1il_ref[...],
                            preferred_element_type=jnp.float32) * INV_LO_SCALE

    def step1(t, h):
        pre = xp1_ref[pl.ds(t, 1), :] + _dot3(h, wt1hh_ref, wt1hl_ref)
        return jnp.tanh(pre)

    h1 = jax.lax.fori_loop(0, CHUNK, step1, hc_ref[1:2, :], unroll=2)
    hc_ref[1:2, :] = h1

    @pl.when(c == NCHUNK - 1)
    def _():
        out_ref[...] = hc_ref[...]


def kernel(x, h, W_ih_0, W_hh_0, b_ih_0, b_hh_0, W_ih_1, W_hh_1, b_ih_1, b_hh_1):
    vmem = lambda: pl.BlockSpec(memory_space=pltpu.VMEM)
    wt0_hi, wt0_lo = _split_hi_lo(W_hh_0.T)
    wt1i_hi, wt1i_lo = _split_hi_lo(W_ih_1.T)
    wt1h_hi, wt1h_lo = _split_hi_lo(W_hh_1.T)
    out = pl.pallas_call(
        _rnn_body,
        grid=(NCHUNK,),
        in_specs=[
            pl.BlockSpec(memory_space=pltpu.SMEM),
            vmem(),
            vmem(),
            vmem(),
            vmem(),
            vmem(), vmem(),
            vmem(), vmem(),
            vmem(),
            vmem(),
            vmem(), vmem(),
        ],
        out_specs=vmem(),
        out_shape=jax.ShapeDtypeStruct((2, HID), jnp.float32),
        scratch_shapes=[
            pltpu.VMEM((CHUNK, HID), jnp.float32),
            pltpu.VMEM((CHUNK, HID), jnp.float32),
            pltpu.VMEM((2, HID), jnp.float32),
        ],
        compiler_params=pltpu.CompilerParams(
            dimension_semantics=("arbitrary",),
            vmem_limit_bytes=100 * 1024 * 1024,
        ),
    )(
        x,
        h.reshape(2, HID),
        W_ih_0.T,
        b_ih_0.reshape(1, HID),
        b_hh_0.reshape(1, HID),
        wt0_hi, wt0_lo,
        wt1i_hi, wt1i_lo,
        b_ih_1.reshape(1, HID),
        b_hh_1.reshape(1, HID),
        wt1h_hi, wt1h_lo,
    )
    return out.reshape(2, 1, HID)


# speedup vs baseline: 4.3067x; 1.0200x over previous
import jax
import jax.numpy as jnp
from jax.experimental import pallas as pl
from jax.experimental.pallas import tpu as pltpu

SEQ_LEN = 8192
HID = 2048
CHUNK = 128
NCHUNK = SEQ_LEN // CHUNK


LO_SCALE = 4096.0
INV_LO_SCALE = 1.0 / LO_SCALE


def _split_hi_lo(w):
    hi = w.astype(jnp.bfloat16)
    lo = ((w - hi.astype(jnp.float32)) * LO_SCALE).astype(jnp.float8_e4m3fn)
    return hi, lo


def _dot3(v, w_hi_ref, w_lo_ref):
    v_hi = v.astype(jnp.bfloat16)
    v_lo = (v - v_hi.astype(jnp.float32)).astype(jnp.bfloat16)
    lhs = jnp.concatenate([v_hi, v_lo], axis=0)
    a = jnp.dot(lhs, w_hi_ref[...], preferred_element_type=jnp.float32)
    b = jnp.dot(v_hi.astype(jnp.float8_e4m3fn), w_lo_ref[...],
                preferred_element_type=jnp.float32)
    return a[0:1, :] + a[1:2, :] + b * INV_LO_SCALE


def _rnn_body(x_ref, h_ref, w0_ref, b0i_ref, b0h_ref,
              wt0h_ref, wt0l_ref, wt1ih_ref, wt1il_ref,
              b1i_ref, b1h_ref, wt1hh_ref, wt1hl_ref,
              out_ref, hs0_ref, xp1_ref, hc_ref):
    c = pl.program_id(0)

    @pl.when(c == 0)
    def _():
        hc_ref[...] = h_ref[...]

    base = c * CHUNK
    w0 = w0_ref[...]
    b0 = b0i_ref[...] + b0h_ref[...]

    def step0(t, h):
        xt = x_ref[0, base + t]
        pre = xt * w0 + b0 + _dot3(h, wt0h_ref, wt0l_ref)
        hn = jnp.tanh(pre)
        hs0_ref[pl.ds(t, 1), :] = hn
        return hn

    h0 = jax.lax.fori_loop(0, CHUNK, step0, hc_ref[0:1, :], unroll=4)
    hc_ref[0:1, :] = h0

    hs = hs0_ref[...]
    hs_hi = hs.astype(jnp.bfloat16)
    xp1_ref[...] = (jnp.dot(hs_hi, wt1ih_ref[...],
                            preferred_element_type=jnp.float32)
                    + b1i_ref[...] + b1h_ref[...])
    hs_lo = (hs - hs_hi.astype(jnp.float32)).astype(jnp.bfloat16)
    xp1_ref[...] += jnp.dot(hs_lo, wt1ih_ref[...],
                            preferred_element_type=jnp.float32)
    xp1_ref[...] += jnp.dot(hs_hi.astype(jnp.float8_e4m3fn), wt1il_ref[...],
                            preferred_element_type=jnp.float32) * INV_LO_SCALE

    def step1(t, h):
        pre = xp1_ref[pl.ds(t, 1), :] + _dot3(h, wt1hh_ref, wt1hl_ref)
        return jnp.tanh(pre)

    h1 = jax.lax.fori_loop(0, CHUNK, step1, hc_ref[1:2, :], unroll=4)
    hc_ref[1:2, :] = h1

    @pl.when(c == NCHUNK - 1)
    def _():
        out_ref[...] = hc_ref[...]


def kernel(x, h, W_ih_0, W_hh_0, b_ih_0, b_hh_0, W_ih_1, W_hh_1, b_ih_1, b_hh_1):
    vmem = lambda: pl.BlockSpec(memory_space=pltpu.VMEM)
    wt0_hi, wt0_lo = _split_hi_lo(W_hh_0.T)
    wt1i_hi, wt1i_lo = _split_hi_lo(W_ih_1.T)
    wt1h_hi, wt1h_lo = _split_hi_lo(W_hh_1.T)
    out = pl.pallas_call(
        _rnn_body,
        grid=(NCHUNK,),
        in_specs=[
            pl.BlockSpec(memory_space=pltpu.SMEM),
            vmem(),
            vmem(),
            vmem(),
            vmem(),
            vmem(), vmem(),
            vmem(), vmem(),
            vmem(),
            vmem(),
            vmem(), vmem(),
        ],
        out_specs=vmem(),
        out_shape=jax.ShapeDtypeStruct((2, HID), jnp.float32),
        scratch_shapes=[
            pltpu.VMEM((CHUNK, HID), jnp.float32),
            pltpu.VMEM((CHUNK, HID), jnp.float32),
            pltpu.VMEM((2, HID), jnp.float32),
        ],
        compiler_params=pltpu.CompilerParams(
            dimension_semantics=("arbitrary",),
            vmem_limit_bytes=100 * 1024 * 1024,
        ),
    )(
        x,
        h.reshape(2, HID),
        W_ih_0.T,
        b_ih_0.reshape(1, HID),
        b_hh_0.reshape(1, HID),
        wt0_hi, wt0_lo,
        wt1i_hi, wt1i_lo,
        b_ih_1.reshape(1, HID),
        b_hh_1.reshape(1, HID),
        wt1h_hi, wt1h_lo,
    )
    return out.reshape(2, 1, HID)


# speedup vs baseline: 4.3090x; 1.0005x over previous
import jax
import jax.numpy as jnp
from jax.experimental import pallas as pl
from jax.experimental.pallas import tpu as pltpu

SEQ_LEN = 8192
HID = 2048
CHUNK = 128
NCHUNK = SEQ_LEN // CHUNK


LO_SCALE = 4096.0
INV_LO_SCALE = 1.0 / LO_SCALE


def _split_hi_lo(w):
    hi = w.astype(jnp.bfloat16)
    lo = ((w - hi.astype(jnp.float32)) * LO_SCALE).astype(jnp.float8_e4m3fn)
    return hi, lo


def _dot3(v, w_hi_ref, w_lo_ref):
    v_hi = v.astype(jnp.bfloat16)
    v_lo = (v - v_hi.astype(jnp.float32)).astype(jnp.bfloat16)
    lhs = jnp.concatenate([v_hi, v_lo], axis=0)
    a = jnp.dot(lhs, w_hi_ref[...], preferred_element_type=jnp.float32)
    b = jnp.dot(v_hi.astype(jnp.float8_e4m3fn), w_lo_ref[...],
                preferred_element_type=jnp.float32)
    return a[0:1, :] + a[1:2, :] + b * INV_LO_SCALE


def _rnn_body(x_ref, h_ref, w0_ref, b0i_ref, b0h_ref,
              wt0h_ref, wt0l_ref, wt1ih_ref, wt1il_ref,
              b1i_ref, b1h_ref, wt1hh_ref, wt1hl_ref,
              out_ref, hs0_ref, xp1_ref, hc_ref):
    c = pl.program_id(0)

    @pl.when(c == 0)
    def _():
        hc_ref[...] = h_ref[...]

    base = c * CHUNK
    w0 = w0_ref[...]
    b0 = b0i_ref[...] + b0h_ref[...]

    def step0(t, h):
        xt = x_ref[0, base + t]
        pre = xt * w0 + b0 + _dot3(h, wt0h_ref, wt0l_ref)
        hn = jnp.tanh(pre)
        hs0_ref[pl.ds(t, 1), :] = hn
        return hn

    def step1(t, h):
        pre = xp1_ref[pl.ds(t, 1), :] + _dot3(h, wt1hh_ref, wt1hl_ref)
        return jnp.tanh(pre)

    @pl.when(c > 0)
    def _gemm():
        hs = hs0_ref[...]
        hs_hi = hs.astype(jnp.bfloat16)
        xp1_ref[...] = (jnp.dot(hs_hi, wt1ih_ref[...],
                                preferred_element_type=jnp.float32)
                        + b1i_ref[...] + b1h_ref[...])
        hs_lo = (hs - hs_hi.astype(jnp.float32)).astype(jnp.bfloat16)
        xp1_ref[...] += jnp.dot(hs_lo, wt1ih_ref[...],
                                preferred_element_type=jnp.float32)
        xp1_ref[...] += jnp.dot(hs_hi.astype(jnp.float8_e4m3fn),
                                wt1il_ref[...],
                                preferred_element_type=jnp.float32) * INV_LO_SCALE

    @pl.when(jnp.logical_and(c > 0, c < NCHUNK))
    def _steady():
        def both(t, hh):
            h0, h1 = hh
            return step0(t, h0), step1(t, h1)

        h0, h1 = jax.lax.fori_loop(
            0, CHUNK, both, (hc_ref[0:1, :], hc_ref[1:2, :]), unroll=2)
        hc_ref[0:1, :] = h0
        hc_ref[1:2, :] = h1

    @pl.when(c == 0)
    def _prologue():
        hc_ref[0:1, :] = jax.lax.fori_loop(
            0, CHUNK, step0, hc_ref[0:1, :], unroll=2)

    @pl.when(c == NCHUNK)
    def _epilogue():
        h1 = jax.lax.fori_loop(0, CHUNK, step1, hc_ref[1:2, :], unroll=2)
        hc_ref[1:2, :] = h1
        out_ref[...] = hc_ref[...]


def kernel(x, h, W_ih_0, W_hh_0, b_ih_0, b_hh_0, W_ih_1, W_hh_1, b_ih_1, b_hh_1):
    vmem = lambda: pl.BlockSpec(memory_space=pltpu.VMEM)
    wt0_hi, wt0_lo = _split_hi_lo(W_hh_0.T)
    wt1i_hi, wt1i_lo = _split_hi_lo(W_ih_1.T)
    wt1h_hi, wt1h_lo = _split_hi_lo(W_hh_1.T)
    out = pl.pallas_call(
        _rnn_body,
        grid=(NCHUNK + 1,),
        in_specs=[
            pl.BlockSpec(memory_space=pltpu.SMEM),
            vmem(),
            vmem(),
            vmem(),
            vmem(),
            vmem(), vmem(),
            vmem(), vmem(),
            vmem(),
            vmem(),
            vmem(), vmem(),
        ],
        out_specs=vmem(),
        out_shape=jax.ShapeDtypeStruct((2, HID), jnp.float32),
        scratch_shapes=[
            pltpu.VMEM((CHUNK, HID), jnp.float32),
            pltpu.VMEM((CHUNK, HID), jnp.float32),
            pltpu.VMEM((2, HID), jnp.float32),
        ],
        compiler_params=pltpu.CompilerParams(
            dimension_semantics=("arbitrary",),
            vmem_limit_bytes=100 * 1024 * 1024,
        ),
    )(
        x,
        h.reshape(2, HID),
        W_ih_0.T,
        b_ih_0.reshape(1, HID),
        b_hh_0.reshape(1, HID),
        wt0_hi, wt0_lo,
        wt1i_hi, wt1i_lo,
        b_ih_1.reshape(1, HID),
        b_hh_1.reshape(1, HID),
        wt1h_hi, wt1h_lo,
    )
    return out.reshape(2, 1, HID)


# speedup vs baseline: 4.3513x; 1.0098x over previous
import jax
import jax.numpy as jnp
from jax.experimental import pallas as pl
from jax.experimental.pallas import tpu as pltpu

SEQ_LEN = 8192
HID = 2048
CHUNK = 128
NCHUNK = SEQ_LEN // CHUNK


LO_SCALE = 4096.0
INV_LO_SCALE = 1.0 / LO_SCALE


def _split_hi_lo(w):
    hi = w.astype(jnp.bfloat16)
    lo = ((w - hi.astype(jnp.float32)) * LO_SCALE).astype(jnp.float8_e4m3fn)
    return hi, lo


def _dot3(v, w_hi_ref, w_lo_ref):
    v_hi = v.astype(jnp.bfloat16)
    v_lo = (v - v_hi.astype(jnp.float32)).astype(jnp.bfloat16)
    lhs = jnp.concatenate([v_hi, v_lo], axis=0)
    a = jnp.dot(lhs, w_hi_ref[...], preferred_element_type=jnp.float32)
    b = jnp.dot(v_hi.astype(jnp.float8_e4m3fn), w_lo_ref[...],
                preferred_element_type=jnp.float32)
    return a[0:1, :] + a[1:2, :] + b * INV_LO_SCALE


def _rnn_body(x_ref, h_ref, w0_ref, b0i_ref, b0h_ref,
              wt0h_ref, wt0l_ref, wt1ih_ref, wt1il_ref,
              b1i_ref, b1h_ref, wt1hh_ref, wt1hl_ref,
              out_ref, hs0_ref, xp1_ref, hc_ref):
    c = pl.program_id(0)

    @pl.when(c == 0)
    def _():
        hc_ref[...] = h_ref[...]

    base = c * CHUNK
    w0 = w0_ref[...]
    b0 = b0i_ref[...] + b0h_ref[...]

    def step0(t, h):
        xt = x_ref[0, base + t]
        pre = xt * w0 + b0 + _dot3(h, wt0h_ref, wt0l_ref)
        hn = jnp.tanh(pre)
        hs0_ref[pl.ds(t, 1), :] = hn
        return hn

    def step1(t, h):
        pre = xp1_ref[pl.ds(t, 1), :] + _dot3(h, wt1hh_ref, wt1hl_ref)
        return jnp.tanh(pre)

    @pl.when(c > 0)
    def _gemm():
        hs = hs0_ref[...]
        hs_hi = hs.astype(jnp.bfloat16)
        xp1_ref[...] = (jnp.dot(hs_hi, wt1ih_ref[...],
                                preferred_element_type=jnp.float32)
                        + b1i_ref[...] + b1h_ref[...])
        hs_lo = (hs - hs_hi.astype(jnp.float32)).astype(jnp.bfloat16)
        xp1_ref[...] += jnp.dot(hs_lo, wt1ih_ref[...],
                                preferred_element_type=jnp.float32)
        xp1_ref[...] += jnp.dot(hs_hi.astype(jnp.float8_e4m3fn),
                                wt1il_ref[...],
                                preferred_element_type=jnp.float32) * INV_LO_SCALE

    @pl.when(jnp.logical_and(c > 0, c < NCHUNK))
    def _steady():
        def both(t, hh):
            h0, h1 = hh
            return step0(t, h0), step1(t, h1)

        h0, h1 = jax.lax.fori_loop(
            0, CHUNK, both, (hc_ref[0:1, :], hc_ref[1:2, :]), unroll=4)
        hc_ref[0:1, :] = h0
        hc_ref[1:2, :] = h1

    @pl.when(c == 0)
    def _prologue():
        hc_ref[0:1, :] = jax.lax.fori_loop(
            0, CHUNK, step0, hc_ref[0:1, :], unroll=4)

    @pl.when(c == NCHUNK)
    def _epilogue():
        h1 = jax.lax.fori_loop(0, CHUNK, step1, hc_ref[1:2, :], unroll=4)
        hc_ref[1:2, :] = h1
        out_ref[...] = hc_ref[...]


def kernel(x, h, W_ih_0, W_hh_0, b_ih_0, b_hh_0, W_ih_1, W_hh_1, b_ih_1, b_hh_1):
    vmem = lambda: pl.BlockSpec(memory_space=pltpu.VMEM)
    wt0_hi, wt0_lo = _split_hi_lo(W_hh_0.T)
    wt1i_hi, wt1i_lo = _split_hi_lo(W_ih_1.T)
    wt1h_hi, wt1h_lo = _split_hi_lo(W_hh_1.T)
    out = pl.pallas_call(
        _rnn_body,
        grid=(NCHUNK + 1,),
        in_specs=[
            pl.BlockSpec(memory_space=pltpu.SMEM),
            vmem(),
            vmem(),
            vmem(),
            vmem(),
            vmem(), vmem(),
            vmem(), vmem(),
            vmem(),
            vmem(),
            vmem(), vmem(),
        ],
        out_specs=vmem(),
        out_shape=jax.ShapeDtypeStruct((2, HID), jnp.float32),
        scratch_shapes=[
            pltpu.VMEM((CHUNK, HID), jnp.float32),
            pltpu.VMEM((CHUNK, HID), jnp.float32),
            pltpu.VMEM((2, HID), jnp.float32),
        ],
        compiler_params=pltpu.CompilerParams(
            dimension_semantics=("arbitrary",),
            vmem_limit_bytes=100 * 1024 * 1024,
        ),
    )(
        x,
        h.reshape(2, HID),
        W_ih_0.T,
        b_ih_0.reshape(1, HID),
        b_hh_0.reshape(1, HID),
        wt0_hi, wt0_lo,
        wt1i_hi, wt1i_lo,
        b_ih_1.reshape(1, HID),
        b_hh_1.reshape(1, HID),
        wt1h_hi, wt1h_lo,
    )
    return out.reshape(2, 1, HID)
